# baseline (speedup 1.0000x reference)
import jax
import jax.numpy as jnp
from jax.experimental import pallas as pl

_NEG = 0.5
_BM = 1000


def _leaky(x):
    return jnp.where(x >= 0, x, _NEG * x)


def _p1(a_ref, e_ref, h_ref):
    i = pl.program_id(0)

    @pl.when(i == 0)
    def _():
        h_ref[...] = jnp.zeros_like(h_ref)

    h_ref[...] += jax.lax.dot_general(
        a_ref[...], e_ref[...], (((0,), (0,)), ((), ())),
        preferred_element_type=jnp.float32)

    @pl.when(i == pl.num_programs(0) - 1)
    def _():
        h_ref[...] = _leaky(h_ref[...])


def _p2(a_ref, h_ref, o_ref):
    o_ref[...] = _leaky(jnp.dot(a_ref[...], h_ref[...],
                                preferred_element_type=jnp.float32))


def kernel(adj, embeds):
    n, e = adj.shape
    d = embeds.shape[1]
    nb = n // _BM
    hids = pl.pallas_call(
        _p1,
        grid=(nb,),
        in_specs=[pl.BlockSpec((_BM, e), lambda i: (i, 0)),
                  pl.BlockSpec((_BM, d), lambda i: (i, 0))],
        out_specs=pl.BlockSpec((e, d), lambda i: (0, 0)),
        out_shape=jax.ShapeDtypeStruct((e, d), jnp.float32),
    )(adj, embeds)
    out = pl.pallas_call(
        _p2,
        grid=(nb,),
        in_specs=[pl.BlockSpec((_BM, e), lambda i: (i, 0)),
                  pl.BlockSpec((e, d), lambda i: (0, 0))],
        out_specs=pl.BlockSpec((_BM, d), lambda i: (i, 0)),
        out_shape=jax.ShapeDtypeStruct((n, d), jnp.float32),
    )(adj, hids)
    return out


# speedup vs baseline: 1.0022x; 1.0022x over previous
import jax
import jax.numpy as jnp
from jax.experimental import pallas as pl
from jax.experimental.pallas import tpu as pltpu

_NEG = 0.5
_BM = 400


def _leaky(x):
    return jnp.where(x >= 0, x, _NEG * x)


def _fused(a_ref, e_ref, o_ref, a_sc, h_sc, hb_sc):
    p = pl.program_id(0)
    i = pl.program_id(1)
    nb = pl.num_programs(1)

    @pl.when(p == 0)
    def _():
        ab = a_ref[...].astype(jnp.bfloat16)
        a_sc[pl.ds(i * _BM, _BM), :] = ab

        @pl.when(i == 0)
        def _():
            h_sc[...] = jnp.zeros_like(h_sc)

        h_sc[...] += jax.lax.dot_general(
            ab, e_ref[...].astype(jnp.bfloat16), (((0,), (0,)), ((), ())),
            preferred_element_type=jnp.float32)

        @pl.when(i == nb - 1)
        def _():
            hb_sc[...] = _leaky(h_sc[...]).astype(jnp.bfloat16)

    @pl.when(p == 1)
    def _():
        o_ref[...] = _leaky(jnp.dot(a_sc[pl.ds(i * _BM, _BM), :], hb_sc[...],
                                    preferred_element_type=jnp.float32))


def kernel(adj, embeds):
    n, e = adj.shape
    d = embeds.shape[1]
    nb = n // _BM
    return pl.pallas_call(
        _fused,
        grid=(2, nb),
        in_specs=[
            pl.BlockSpec((_BM, e), lambda p, i: (i * (1 - p) + (nb - 1) * p, 0)),
            pl.BlockSpec((_BM, d), lambda p, i: (i * (1 - p) + (nb - 1) * p, 0)),
        ],
        out_specs=pl.BlockSpec((_BM, d), lambda p, i: (i * p, 0)),
        out_shape=jax.ShapeDtypeStruct((n, d), jnp.float32),
        scratch_shapes=[
            pltpu.VMEM((n, e), jnp.bfloat16),
            pltpu.VMEM((e, d), jnp.float32),
            pltpu.VMEM((e, d), jnp.bfloat16),
        ],
    )(adj, embeds)


# speedup vs baseline: 1.0892x; 1.0868x over previous
import jax
import jax.numpy as jnp
from jax.experimental import pallas as pl
from jax.experimental.pallas import tpu as pltpu

_NEG = 0.5
_BM0 = 400
_BM1 = 2000


def _leaky(x):
    return jnp.where(x >= 0, x, _NEG * x)


def _fused(a_ref, e_ref, o_ref, a_sc, h_sc, hb_sc, *, nb0):
    i = pl.program_id(0)

    @pl.when(i < nb0)
    def _():
        ab = a_ref[...].astype(jnp.bfloat16)
        a_sc[pl.ds(i * _BM0, _BM0), :] = ab

        @pl.when(i == 0)
        def _():
            h_sc[...] = jnp.zeros_like(h_sc)

        h_sc[...] += jax.lax.dot_general(
            ab, e_ref[...].astype(jnp.bfloat16), (((0,), (0,)), ((), ())),
            preferred_element_type=jnp.float32)

        @pl.when(i == nb0 - 1)
        def _():
            hb_sc[...] = _leaky(h_sc[...]).astype(jnp.bfloat16)

    @pl.when(i >= nb0)
    def _():
        j = i - nb0
        o_ref[...] = _leaky(jnp.dot(a_sc[pl.ds(j * _BM1, _BM1), :],
                                    hb_sc[...],
                                    preferred_element_type=jnp.float32))


def kernel(adj, embeds):
    n, e = adj.shape
    d = embeds.shape[1]
    nb0 = n // _BM0
    nb1 = n // _BM1
    import functools
    body = functools.partial(_fused, nb0=nb0)
    return pl.pallas_call(
        body,
        grid=(nb0 + nb1,),
        in_specs=[
            pl.BlockSpec((_BM0, e), lambda i: (jnp.minimum(i, nb0 - 1), 0)),
            pl.BlockSpec((_BM0, d), lambda i: (jnp.minimum(i, nb0 - 1), 0)),
        ],
        out_specs=pl.BlockSpec((_BM1, d), lambda i: (jnp.maximum(i - nb0, 0), 0)),
        out_shape=jax.ShapeDtypeStruct((n, d), jnp.float32),
        scratch_shapes=[
            pltpu.VMEM((n, e), jnp.bfloat16),
            pltpu.VMEM((e, d), jnp.float32),
            pltpu.VMEM((e, d), jnp.bfloat16),
        ],
    )(adj, embeds)
